# baseline (speedup 1.0000x reference)
import functools

import jax
import jax.numpy as jnp
from jax import lax
from jax.experimental import pallas as pl
from jax.experimental.pallas import tpu as pltpu
from jax.experimental.pallas import tpu_sc as plsc

BN_EPS = 1e-5
_IDX_W = 128


def _make_sc_gather(num_rows, D, NW, b_per_w):
    K = b_per_w // _IDX_W
    mesh = plsc.VectorSubcoreMesh(core_axis_name="c", subcore_axis_name="s")

    @functools.partial(
        pl.kernel,
        mesh=mesh,
        compiler_params=pltpu.CompilerParams(use_tc_tiling_on_sc=False),
        out_type=jax.ShapeDtypeStruct((num_rows, D), jnp.float32),
        scratch_types=[
            pltpu.VMEM((K, _IDX_W), jnp.int32),
            pltpu.VMEM((b_per_w, D), jnp.float32),
            pltpu.SemaphoreType.DMA,
        ],
    )
    def gather_k(idx_hbm, table_hbm, out_hbm, idx_v, rows_v, sem):
        nc = 2
        wid = lax.axis_index("s") * nc + lax.axis_index("c")
        base = wid * b_per_w
        pltpu.sync_copy(idx_hbm.at[wid], idx_v)
        copies = [
            pltpu.async_copy(
                table_hbm.at[idx_v.at[j]],
                rows_v.at[pl.ds(j * _IDX_W, _IDX_W)],
                sem,
            )
            for j in range(K)
        ]
        for c in copies:
            c.wait()
        pltpu.sync_copy(rows_v, out_hbm.at[pl.ds(base, b_per_w)])

    return gather_k


def _tc_body(g_ref, rels_ref, gamma_ref, beta_ref, relw_t_ref, loss_ref, preds_ref):
    B = rels_ref.shape[0]
    R = preds_ref.shape[1]

    def encode(x):
        n = jnp.sqrt(jnp.sum(x * x, axis=1, keepdims=True))
        x = jnp.where(n > 1.0, x / (n + 1e-7), x)
        mean = jnp.mean(x, axis=0, keepdims=True)
        var = jnp.mean((x - mean) ** 2, axis=0, keepdims=True)
        x = (x - mean) / jnp.sqrt(var + BN_EPS)
        return x * gamma_ref[...] + beta_ref[...]

    h = encode(g_ref[:B, :])
    t = encode(g_ref[B:, :])
    logits = jnp.dot(h * t, relw_t_ref[...], preferred_element_type=jnp.float32)
    m = jnp.max(logits, axis=1, keepdims=True)
    ex = jnp.exp(logits - m)
    s = jnp.sum(ex, axis=1, keepdims=True)
    lp = logits - m - jnp.log(s)
    preds_ref[...] = ex / s
    onehot = lax.broadcasted_iota(jnp.int32, (B, R), 1) == rels_ref[...]
    picked = jnp.sum(jnp.where(onehot, lp, 0.0), axis=0, keepdims=True)
    loss_ref[...] = -jnp.sum(picked, axis=1, keepdims=True) / B


def kernel(pos_edges, emb_table, bn_gamma, bn_beta, rel_w):
    B = pos_edges.shape[0]
    V, D = emb_table.shape
    R = rel_w.shape[0]

    NW = 32
    num_rows = 2 * B
    b_per_w = num_rows // NW

    idx = jnp.concatenate([pos_edges[:, 0], pos_edges[:, 2]], axis=0)
    idx3 = idx.reshape(NW, b_per_w // _IDX_W, _IDX_W)

    gathered = _make_sc_gather(num_rows, D, NW, b_per_w)(idx3, emb_table)

    rels2d = pos_edges[:, 1].reshape(B, 1)
    gamma2d = bn_gamma.reshape(1, D)
    beta2d = bn_beta.reshape(1, D)
    relw_t = rel_w.T

    loss2d, preds = pl.pallas_call(
        _tc_body,
        out_shape=[
            jax.ShapeDtypeStruct((1, 1), jnp.float32),
            jax.ShapeDtypeStruct((B, R), jnp.float32),
        ],
    )(gathered, rels2d, gamma2d, beta2d, relw_t)

    return (loss2d[0, 0], preds)
